# baseline (speedup 1.0000x reference)
import functools

import jax
import jax.numpy as jnp
from jax import lax
from jax.experimental import pallas as pl
from jax.experimental.pallas import tpu as pltpu
from jax.experimental.pallas import tpu_sc as plsc

B = 1024
N_CHANS = 64
WINDOW = 1000
N_OUT = 4
E = 16
FEAT = 512

WB = 40
NSTEP = WINDOW // WB


def _tc_body(xT_ref, Wbb_ref, bbb_ref, Wall_ref, ball_ref,
             allh_ref, acc_ref):
    i = pl.program_id(0)

    @pl.when(i == 0)
    def _():
        acc_ref[...] = jnp.zeros_like(acc_ref)

    acc_ref[...] += jnp.sum(xT_ref[...], axis=1)

    @pl.when(i == NSTEP - 1)
    def _():
        m = acc_ref[...] * (1.0 / WINDOW)
        dn = (((0,), (0,)), ((), ()))
        featsT = jax.lax.dot_general(Wbb_ref[...], m, dn,
                                     preferred_element_type=jnp.float32)
        featsT = jnp.maximum(featsT + bbb_ref[...], 0.0)
        allh = jax.lax.dot_general(featsT, Wall_ref[...], dn,
                                   preferred_element_type=jnp.float32)
        allh_ref[...] = allh + ball_ref[...]


SC_STRIPE = 128


def _sc_route_body(allh_hbm, sid_hbm, out_hbm, allh_v, sid_v, out_v, nc):
    wid = lax.axis_index("s") * nc + lax.axis_index("c")
    g = wid // N_OUT
    o = wid % N_OUT
    base = g * SC_STRIPE
    pltpu.sync_copy(allh_hbm.at[pl.ds(base * (E * N_OUT),
                                      SC_STRIPE * (E * N_OUT))], allh_v)
    pltpu.sync_copy(sid_hbm.at[pl.ds(base, SC_STRIPE)], sid_v)
    lanes = jax.lax.iota(jnp.int32, 16)
    for h in range(SC_STRIPE // 16):
        sidvec = sid_v[pl.ds(h * 16, 16)]
        idx = (lanes + h * 16) * (E * N_OUT) + sidvec * N_OUT + o
        val = plsc.load_gather(allh_v, [idx])
        out_v[pl.ds(h * 16, 16)] = val
    pltpu.sync_copy(out_v, out_hbm.at[pl.ds(o * B + base, SC_STRIPE)])


@jax.jit
def kernel(x, subject_ids, W_bb, b_bb, W_heads, b_heads):
    xT = jnp.transpose(x, (1, 2, 0))
    sid = subject_ids.astype(jnp.int32)
    W_all = W_heads.transpose(1, 0, 2).reshape(FEAT, E * N_OUT)
    b_all = b_heads.reshape(1, E * N_OUT)
    bbb = b_bb.reshape(FEAT, 1)

    allh = pl.pallas_call(
        _tc_body,
        grid=(NSTEP,),
        in_specs=[
            pl.BlockSpec((N_CHANS, WB, B), lambda i: (0, i, 0)),
            pl.BlockSpec((N_CHANS, FEAT), lambda i: (0, 0)),
            pl.BlockSpec((FEAT, 1), lambda i: (0, 0)),
            pl.BlockSpec((FEAT, E * N_OUT), lambda i: (0, 0)),
            pl.BlockSpec((1, E * N_OUT), lambda i: (0, 0)),
        ],
        out_specs=pl.BlockSpec((B, E * N_OUT), lambda i: (0, 0)),
        out_shape=jax.ShapeDtypeStruct((B, E * N_OUT), jnp.float32),
        scratch_shapes=[pltpu.VMEM((N_CHANS, B), jnp.float32)],
    )(xT, W_bb, bbb, W_all, b_all)
    allh_flat = allh.reshape(B * E * N_OUT)

    info = plsc.get_sparse_core_info()
    nc = info.num_cores
    mesh = plsc.VectorSubcoreMesh(core_axis_name="c", subcore_axis_name="s")
    sc_route = pl.kernel(
        functools.partial(_sc_route_body, nc=nc),
        mesh=mesh,
        compiler_params=pltpu.CompilerParams(use_tc_tiling_on_sc=False, needs_layout_passes=False),
        out_type=jax.ShapeDtypeStruct((N_OUT * B,), jnp.float32),
        scratch_types=[
            pltpu.VMEM((SC_STRIPE * E * N_OUT,), jnp.float32),
            pltpu.VMEM((SC_STRIPE,), jnp.int32),
            pltpu.VMEM((SC_STRIPE,), jnp.float32),
        ],
    )
    out_flat = sc_route(allh_flat, sid)
    return out_flat.reshape(N_OUT, B).T
